# baseline (speedup 1.0000x reference)
import jax
import jax.numpy as jnp
from jax import lax
from jax.experimental import pallas as pl
from jax.experimental.pallas import tpu as pltpu
from jax.experimental.pallas import tpu_sc as plsc

N = 10000
E = 160000
T = 12
F = 32
D = T * F
DQ = D // 4

NS = 16
CH = 40
LCH = 256
NBUF = 2
EP = NS * CH * LCH
NPAD = 10112
SLAB = NPAD // NS
BN = 400
BNP = 632
GRID_N = N // BN
GRID_P = NPAD // BNP

_sc_mesh = plsc.VectorSubcoreMesh(core_axis_name="c", subcore_axis_name="s")
_sc_params = pltpu.CompilerParams(use_tc_tiling_on_sc=False)


def _deg_body(srcI, dstI, ones_h, z8, degS, degD, idx_v, ones_v, deg_sh):
    c = lax.axis_index("c")
    s = lax.axis_index("s")
    sl = pl.ds(s * SLAB, SLAB)
    pltpu.sync_copy(z8, deg_sh.at[sl])
    pltpu.sync_copy(ones_h, ones_v)

    @pl.when(c == 0)
    def _():
        pltpu.sync_copy(srcI.at[s], idx_v)

    @pl.when(c == 1)
    def _():
        pltpu.sync_copy(dstI.at[s], idx_v)

    plsc.subcore_barrier()

    def body(j, carry):
        pltpu.sync_copy(ones_v, deg_sh.at[idx_v.at[j]], add=True)
        return carry

    lax.fori_loop(0, CH, body, 0)
    plsc.subcore_barrier()

    @pl.when(c == 0)
    def _():
        pltpu.sync_copy(deg_sh.at[sl], degS.at[sl])

    @pl.when(c == 1)
    def _():
        pltpu.sync_copy(deg_sh.at[sl], degD.at[sl])


_deg_call = pl.kernel(
    _deg_body,
    out_type=(
        jax.ShapeDtypeStruct((NPAD, 8), jnp.float32),
        jax.ShapeDtypeStruct((NPAD, 8), jnp.float32),
    ),
    mesh=_sc_mesh,
    scratch_types=[
        pltpu.VMEM((CH, LCH), jnp.int32),
        pltpu.VMEM((LCH, 8), jnp.float32),
        pltpu.VMEM_SHARED((NPAD, 8), jnp.float32),
    ],
    compiler_params=_sc_params,
)


def _agg_body(t0, t1, t2, t3, srcI, dstI, zrows, a0, a1, a2, a3,
              src_v, dst_v, rows, agg_sh, gsems, ssems):
    c = lax.axis_index("c")
    s = lax.axis_index("s")
    sl = pl.ds(s * SLAB, SLAB)
    pltpu.sync_copy(srcI.at[s], src_v)
    pltpu.sync_copy(dstI.at[s], dst_v)

    def quarter(tab, aout):
        pltpu.sync_copy(zrows, agg_sh.at[sl])
        plsc.subcore_barrier()
        pltpu.async_copy(tab.at[src_v.at[0]], rows.at[0], gsems.at[0])

        def body(i, carry):
            for b in range(NBUF):
                j = NBUF * i + b
                bn = (b + 1) % NBUF
                pltpu.make_async_copy(
                    tab.at[src_v.at[j]], rows.at[b], gsems.at[b]).wait()
                pltpu.async_copy(rows.at[b], agg_sh.at[dst_v.at[j]],
                                 ssems.at[b], add=True)

                @pl.when(j + 1 < CH)
                def _():
                    @pl.when(j >= 1)
                    def _():
                        pltpu.make_async_copy(
                            rows.at[bn], agg_sh.at[dst_v.at[j - 1]],
                            ssems.at[bn]).wait()

                    pltpu.async_copy(tab.at[src_v.at[j + 1]],
                                     rows.at[bn], gsems.at[bn])

            return carry

        lax.fori_loop(0, CH // NBUF, body, 0)
        for q in range(CH - 2, CH):
            pltpu.make_async_copy(
                rows.at[q % NBUF], agg_sh.at[dst_v.at[q]],
                ssems.at[q % NBUF]).wait()
        plsc.subcore_barrier()
        pltpu.sync_copy(agg_sh.at[sl], aout.at[sl])

    @pl.when(c == 0)
    def _():
        quarter(t0, a0)
        quarter(t1, a1)

    @pl.when(c == 1)
    def _():
        quarter(t2, a2)
        quarter(t3, a3)


_agg_call = pl.kernel(
    _agg_body,
    out_type=tuple(jax.ShapeDtypeStruct((NPAD, DQ), jnp.float32)
                   for _ in range(4)),
    mesh=_sc_mesh,
    scratch_types=[
        pltpu.VMEM((CH, LCH), jnp.int32),
        pltpu.VMEM((CH, LCH), jnp.int32),
        pltpu.VMEM((NBUF, LCH, DQ), jnp.float32),
        pltpu.VMEM_SHARED((NPAD, DQ), jnp.float32),
        pltpu.SemaphoreType.DMA((NBUF,)),
        pltpu.SemaphoreType.DMA((NBUF,)),
    ],
    compiler_params=_sc_params,
)


def _store_quarters(hw, orefs):
    for q, oref in enumerate(orefs):
        oref[...] = hw[:, q * DQ:(q + 1) * DQ]


def _first_body(x_ref, dS_ref, w_ref, o0_ref, o1_ref, o2_ref, o3_ref):
    sn = lax.rsqrt(jnp.maximum(dS_ref[:, :1], 1.0))
    hw = jnp.dot(x_ref[...], w_ref[...],
                 preferred_element_type=jnp.float32) * sn
    _store_quarters(hw, (o0_ref, o1_ref, o2_ref, o3_ref))


def _mid_body(a0_ref, a1_ref, a2_ref, a3_ref, dD_ref, dS_ref, w_ref, b_ref,
              o0_ref, o1_ref, o2_ref, o3_ref):
    agg = jnp.concatenate(
        [a0_ref[...], a1_ref[...], a2_ref[...], a3_ref[...]], axis=1)
    dn = lax.rsqrt(jnp.maximum(dD_ref[:, :1], 1.0))
    h = jnp.maximum(agg * dn + b_ref[...], 0.0)
    sn = lax.rsqrt(jnp.maximum(dS_ref[:, :1], 1.0))
    hw = jnp.dot(h, w_ref[...], preferred_element_type=jnp.float32) * sn
    _store_quarters(hw, (o0_ref, o1_ref, o2_ref, o3_ref))


def _last_body(a0_ref, a1_ref, a2_ref, a3_ref, dD_ref, b_ref, wt_ref, cb_ref,
               o_ref):
    agg = jnp.concatenate(
        [a0_ref[...], a1_ref[...], a2_ref[...], a3_ref[...]], axis=1)
    dn = lax.rsqrt(jnp.maximum(dD_ref[:, :1], 1.0))
    h = agg * dn + b_ref[...]
    o_ref[...] = jnp.dot(h, wt_ref[...],
                         preferred_element_type=jnp.float32) + cb_ref[...]


def _row_spec(bn, width):
    return pl.BlockSpec((bn, width), lambda i: (i, 0))


_wbig_spec = pl.BlockSpec((D, D), lambda i: (0, 0))
_b_spec = pl.BlockSpec((1, D), lambda i: (0, 0))


def _quarter_shapes(n):
    return tuple(jax.ShapeDtypeStruct((n, DQ), jnp.float32) for _ in range(4))


def _first_call(x, degS, Wbig):
    return pl.pallas_call(
        _first_body,
        grid=(GRID_N,),
        in_specs=[_row_spec(BN, D), _row_spec(BN, 8), _wbig_spec],
        out_specs=tuple(_row_spec(BN, DQ) for _ in range(4)),
        out_shape=_quarter_shapes(N),
    )(x, degS, Wbig)


def _mid_call(a, degD, degS, Wbig, bvec):
    return pl.pallas_call(
        _mid_body,
        grid=(GRID_P,),
        in_specs=[_row_spec(BNP, DQ)] * 4 + [_row_spec(BNP, 8),
                                             _row_spec(BNP, 8),
                                             _wbig_spec, _b_spec],
        out_specs=tuple(_row_spec(BNP, DQ) for _ in range(4)),
        out_shape=_quarter_shapes(NPAD),
    )(*a, degD, degS, Wbig, bvec)


def _last_call(a, degD, bvec, WT, cb):
    return pl.pallas_call(
        _last_body,
        grid=(GRID_P,),
        in_specs=[_row_spec(BNP, DQ)] * 4 + [_row_spec(BNP, 8), _b_spec,
                                             _wbig_spec, _b_spec],
        out_specs=_row_spec(BNP, D),
        out_shape=jax.ShapeDtypeStruct((NPAD, D), jnp.float32),
    )(*a, degD, bvec, WT, cb)


@jax.jit
def kernel(temporal_features, edge_index, W1, b1, W2, b2, W3, b3, conv_w, conv_b):
    x = jnp.transpose(temporal_features, (0, 2, 1)).reshape(N, D)
    src = edge_index[0]
    dst = edge_index[1]
    pad = EP - E
    srcA = jnp.concatenate([src, jnp.zeros((pad,), jnp.int32)]).reshape(NS, CH, LCH)
    srcD = jnp.concatenate([src, jnp.full((pad,), N, jnp.int32)]).reshape(NS, CH, LCH)
    dstI = jnp.concatenate([dst, jnp.full((pad,), N, jnp.int32)]).reshape(NS, CH, LCH)

    ones8 = jnp.ones((LCH, 8), jnp.float32)
    z8 = jnp.zeros((SLAB, 8), jnp.float32)
    zrows = jnp.zeros((SLAB, DQ), jnp.float32)

    eyeT = jnp.eye(T, dtype=jnp.float32)
    W1big = jnp.kron(eyeT, W1)
    W2big = jnp.kron(eyeT, W2)
    W3big = jnp.kron(eyeT, W3)
    WT = sum(jnp.kron(jnp.eye(T, k=1 - k, dtype=jnp.float32),
                      conv_w[:, :, k].T) for k in range(3))
    b1t = jnp.tile(b1, T).reshape(1, D)
    b2t = jnp.tile(b2, T).reshape(1, D)
    b3t = jnp.tile(b3, T).reshape(1, D)
    cbt = jnp.tile(conv_b, T).reshape(1, D)

    degS, degD = _deg_call(srcD, dstI, ones8, z8)

    def agg(ts):
        a = _agg_call(*ts, srcA, dstI, zrows)
        return a

    t = _first_call(x, degS[:N], W1big)
    a = agg(t)
    t = _mid_call(a, degD, degS, W2big, b1t)
    a = agg(t)
    t = _mid_call(a, degD, degS, W3big, b2t)
    a = agg(t)

    y = _last_call(a, degD, b3t, WT, cbt)
    return jnp.transpose(y[:N].reshape(N, T, F), (0, 2, 1))


# speedup vs baseline: 1.0370x; 1.0370x over previous
import jax
import jax.numpy as jnp
from jax import lax
from jax.experimental import pallas as pl
from jax.experimental.pallas import tpu as pltpu
from jax.experimental.pallas import tpu_sc as plsc

N = 10000
E = 160000
T = 12
F = 32
D = T * F
DQ = D // 4

NS = 16
CH = 40
LCH = 256
NBUF = 2
EP = NS * CH * LCH
NPAD = 10112
SLAB = NPAD // NS
BN = 400
BNP = 632
GRID_N = N // BN
GRID_P = NPAD // BNP

_sc_mesh = plsc.VectorSubcoreMesh(core_axis_name="c", subcore_axis_name="s")
_sc_params = pltpu.CompilerParams(use_tc_tiling_on_sc=False)


def _deg_body(srcI, dstI, ones_h, z8, degS, degD, idx_v, ones_v, deg_sh):
    c = lax.axis_index("c")
    s = lax.axis_index("s")
    sl = pl.ds(s * SLAB, SLAB)
    pltpu.sync_copy(z8, deg_sh.at[sl])
    pltpu.sync_copy(ones_h, ones_v)

    @pl.when(c == 0)
    def _():
        pltpu.sync_copy(srcI.at[s], idx_v)

    @pl.when(c == 1)
    def _():
        pltpu.sync_copy(dstI.at[s], idx_v)

    plsc.subcore_barrier()

    def body(j, carry):
        pltpu.sync_copy(ones_v, deg_sh.at[idx_v.at[j]], add=True)
        return carry

    lax.fori_loop(0, CH, body, 0)
    plsc.subcore_barrier()

    @pl.when(c == 0)
    def _():
        pltpu.sync_copy(deg_sh.at[sl], degS.at[sl])

    @pl.when(c == 1)
    def _():
        pltpu.sync_copy(deg_sh.at[sl], degD.at[sl])


_deg_call = pl.kernel(
    _deg_body,
    out_type=(
        jax.ShapeDtypeStruct((NPAD, 8), jnp.float32),
        jax.ShapeDtypeStruct((NPAD, 8), jnp.float32),
    ),
    mesh=_sc_mesh,
    scratch_types=[
        pltpu.VMEM((CH, LCH), jnp.int32),
        pltpu.VMEM((LCH, 8), jnp.float32),
        pltpu.VMEM_SHARED((NPAD, 8), jnp.float32),
    ],
    compiler_params=_sc_params,
)


def _agg_body(t0, t1, t2, t3, srcI, dstI, zrows, a0, a1, a2, a3,
              src_v, dst_v, rows, agg_sh, gsems, ssems):
    c = lax.axis_index("c")
    s = lax.axis_index("s")
    sl = pl.ds(s * SLAB, SLAB)
    pltpu.sync_copy(srcI.at[s], src_v)
    pltpu.sync_copy(dstI.at[s], dst_v)

    def quarter(tab, aout):
        pltpu.sync_copy(zrows, agg_sh.at[sl])
        plsc.subcore_barrier()
        for b in range(NBUF):
            pltpu.async_copy(tab.at[src_v.at[b]], rows.at[b], gsems.at[b])

        def body(i, carry):
            for b in range(NBUF):
                j = NBUF * i + b

                @pl.when(j < CH)
                def _():
                    pltpu.make_async_copy(
                        tab.at[src_v.at[j]], rows.at[b], gsems.at[b]).wait()
                    pltpu.async_copy(rows.at[b], agg_sh.at[dst_v.at[j]],
                                     ssems.at[b], add=True)

                    @pl.when(j + NBUF < CH)
                    def _():
                        pltpu.make_async_copy(
                            rows.at[b], agg_sh.at[dst_v.at[j]],
                            ssems.at[b]).wait()
                        pltpu.async_copy(tab.at[src_v.at[j + NBUF]],
                                         rows.at[b], gsems.at[b])

            return carry

        lax.fori_loop(0, (CH + NBUF - 1) // NBUF, body, 0)
        for b in range(NBUF):
            pltpu.make_async_copy(
                rows.at[b], agg_sh.at[dst_v.at[CH - NBUF + b]],
                ssems.at[b]).wait()
        plsc.subcore_barrier()
        pltpu.sync_copy(agg_sh.at[sl], aout.at[sl])

    @pl.when(c == 0)
    def _():
        quarter(t0, a0)
        quarter(t1, a1)

    @pl.when(c == 1)
    def _():
        quarter(t2, a2)
        quarter(t3, a3)


_agg_call = pl.kernel(
    _agg_body,
    out_type=tuple(jax.ShapeDtypeStruct((NPAD, DQ), jnp.float32)
                   for _ in range(4)),
    mesh=_sc_mesh,
    scratch_types=[
        pltpu.VMEM((CH, LCH), jnp.int32),
        pltpu.VMEM((CH, LCH), jnp.int32),
        pltpu.VMEM((NBUF, LCH, DQ), jnp.float32),
        pltpu.VMEM_SHARED((NPAD, DQ), jnp.float32),
        pltpu.SemaphoreType.DMA((NBUF,)),
        pltpu.SemaphoreType.DMA((NBUF,)),
    ],
    compiler_params=_sc_params,
)


def _store_quarters(hw, orefs):
    for q, oref in enumerate(orefs):
        oref[...] = hw[:, q * DQ:(q + 1) * DQ]


def _first_body(x_ref, dS_ref, w_ref, o0_ref, o1_ref, o2_ref, o3_ref):
    sn = lax.rsqrt(jnp.maximum(dS_ref[:, :1], 1.0))
    hw = jnp.dot(x_ref[...], w_ref[...],
                 preferred_element_type=jnp.float32) * sn
    _store_quarters(hw, (o0_ref, o1_ref, o2_ref, o3_ref))


def _mid_body(a0_ref, a1_ref, a2_ref, a3_ref, dD_ref, dS_ref, w_ref, b_ref,
              o0_ref, o1_ref, o2_ref, o3_ref):
    agg = jnp.concatenate(
        [a0_ref[...], a1_ref[...], a2_ref[...], a3_ref[...]], axis=1)
    dn = lax.rsqrt(jnp.maximum(dD_ref[:, :1], 1.0))
    h = jnp.maximum(agg * dn + b_ref[...], 0.0)
    sn = lax.rsqrt(jnp.maximum(dS_ref[:, :1], 1.0))
    hw = jnp.dot(h, w_ref[...], preferred_element_type=jnp.float32) * sn
    _store_quarters(hw, (o0_ref, o1_ref, o2_ref, o3_ref))


def _last_body(a0_ref, a1_ref, a2_ref, a3_ref, dD_ref, b_ref, wt_ref, cb_ref,
               o_ref):
    agg = jnp.concatenate(
        [a0_ref[...], a1_ref[...], a2_ref[...], a3_ref[...]], axis=1)
    dn = lax.rsqrt(jnp.maximum(dD_ref[:, :1], 1.0))
    h = agg * dn + b_ref[...]
    o_ref[...] = jnp.dot(h, wt_ref[...],
                         preferred_element_type=jnp.float32) + cb_ref[...]


def _row_spec(bn, width):
    return pl.BlockSpec((bn, width), lambda i: (i, 0))


_wbig_spec = pl.BlockSpec((D, D), lambda i: (0, 0))
_b_spec = pl.BlockSpec((1, D), lambda i: (0, 0))


def _quarter_shapes(n):
    return tuple(jax.ShapeDtypeStruct((n, DQ), jnp.float32) for _ in range(4))


def _first_call(x, degS, Wbig):
    return pl.pallas_call(
        _first_body,
        grid=(GRID_N,),
        in_specs=[_row_spec(BN, D), _row_spec(BN, 8), _wbig_spec],
        out_specs=tuple(_row_spec(BN, DQ) for _ in range(4)),
        out_shape=_quarter_shapes(N),
    )(x, degS, Wbig)


def _mid_call(a, degD, degS, Wbig, bvec):
    return pl.pallas_call(
        _mid_body,
        grid=(GRID_P,),
        in_specs=[_row_spec(BNP, DQ)] * 4 + [_row_spec(BNP, 8),
                                             _row_spec(BNP, 8),
                                             _wbig_spec, _b_spec],
        out_specs=tuple(_row_spec(BNP, DQ) for _ in range(4)),
        out_shape=_quarter_shapes(NPAD),
    )(*a, degD, degS, Wbig, bvec)


def _last_call(a, degD, bvec, WT, cb):
    return pl.pallas_call(
        _last_body,
        grid=(GRID_P,),
        in_specs=[_row_spec(BNP, DQ)] * 4 + [_row_spec(BNP, 8), _b_spec,
                                             _wbig_spec, _b_spec],
        out_specs=_row_spec(BNP, D),
        out_shape=jax.ShapeDtypeStruct((NPAD, D), jnp.float32),
    )(*a, degD, bvec, WT, cb)


@jax.jit
def kernel(temporal_features, edge_index, W1, b1, W2, b2, W3, b3, conv_w, conv_b):
    x = jnp.transpose(temporal_features, (0, 2, 1)).reshape(N, D)
    src = edge_index[0]
    dst = edge_index[1]
    pad = EP - E
    srcA = jnp.concatenate([src, jnp.zeros((pad,), jnp.int32)]).reshape(NS, CH, LCH)
    srcD = jnp.concatenate([src, jnp.full((pad,), N, jnp.int32)]).reshape(NS, CH, LCH)
    dstI = jnp.concatenate([dst, jnp.full((pad,), N, jnp.int32)]).reshape(NS, CH, LCH)

    ones8 = jnp.ones((LCH, 8), jnp.float32)
    z8 = jnp.zeros((SLAB, 8), jnp.float32)
    zrows = jnp.zeros((SLAB, DQ), jnp.float32)

    eyeT = jnp.eye(T, dtype=jnp.float32)
    W1big = jnp.kron(eyeT, W1)
    W2big = jnp.kron(eyeT, W2)
    W3big = jnp.kron(eyeT, W3)
    WT = sum(jnp.kron(jnp.eye(T, k=1 - k, dtype=jnp.float32),
                      conv_w[:, :, k].T) for k in range(3))
    b1t = jnp.tile(b1, T).reshape(1, D)
    b2t = jnp.tile(b2, T).reshape(1, D)
    b3t = jnp.tile(b3, T).reshape(1, D)
    cbt = jnp.tile(conv_b, T).reshape(1, D)

    degS, degD = _deg_call(srcD, dstI, ones8, z8)

    def agg(ts):
        a = _agg_call(*ts, srcA, dstI, zrows)
        return a

    t = _first_call(x, degS[:N], W1big)
    a = agg(t)
    t = _mid_call(a, degD, degS, W2big, b1t)
    a = agg(t)
    t = _mid_call(a, degD, degS, W3big, b2t)
    a = agg(t)

    y = _last_call(a, degD, b3t, WT, cbt)
    return jnp.transpose(y[:N].reshape(N, T, F), (0, 2, 1))
